# baseline (device time: 24479 ns/iter reference)
import jax
import jax.numpy as jnp
from jax import lax
from jax.experimental import pallas as pl
from jax.experimental.pallas import tpu as pltpu

N_CHUNKS = 8


def kernel(A, B):
    M, Ks = A.shape
    _, N = B.shape
    HALF = M // 2
    CH = HALF // N_CHUNKS

    def body(a_ref, b_ref, out_ref, b_bf, xsend_buf, xrecv_buf,
             ysend_buf, yrecv_buf,
             xsend_sems, xrecv_sems, ysend_sems, yrecv_sems):
        my_x = lax.axis_index("x")
        my_y = lax.axis_index("y")
        xpeer = (1 - my_x, my_y)
        ypeer = (my_x, 1 - my_y)
        base = my_y * HALF
        obase = (1 - my_y) * HALF

        barrier_sem = pltpu.get_barrier_semaphore()
        for peer in (xpeer, ypeer):
            pl.semaphore_signal(
                barrier_sem, inc=1, device_id=peer,
                device_id_type=pl.DeviceIdType.MESH,
            )
        pl.semaphore_wait(barrier_sem, 2)

        def x_rdma(c):
            return pltpu.make_async_remote_copy(
                src_ref=xsend_buf.at[c],
                dst_ref=xrecv_buf.at[c],
                send_sem=xsend_sems.at[c],
                recv_sem=xrecv_sems.at[c],
                device_id=xpeer,
                device_id_type=pl.DeviceIdType.MESH,
            )

        def y_rdma(c):
            return pltpu.make_async_remote_copy(
                src_ref=ysend_buf.at[c],
                dst_ref=yrecv_buf.at[c],
                send_sem=ysend_sems.at[c],
                recv_sem=yrecv_sems.at[c],
                device_id=ypeer,
                device_id_type=pl.DeviceIdType.MESH,
            )

        b_bf[:, :] = b_ref[:, :].astype(jnp.bfloat16)

        def compute_send(c):
            rows = pl.ds(base + c * CH, CH)
            p = jnp.dot(
                a_ref[rows, :].astype(jnp.bfloat16), b_bf[:, :],
                preferred_element_type=jnp.float32,
            )
            out_ref[rows, :] = p
            xsend_buf[c] = p.astype(jnp.bfloat16)
            x_rdma(c).start()

        def reduce_forward(c):
            x_rdma(c).wait_recv()
            rows = pl.ds(base + c * CH, CH)
            red = out_ref[rows, :] + xrecv_buf[c].astype(jnp.float32)
            out_ref[rows, :] = red
            ysend_buf[c] = red.astype(jnp.bfloat16)
            y_rdma(c).start()

        def store_other_half(c):
            y_rdma(c).wait_recv()
            orows = pl.ds(obase + c * CH, CH)
            out_ref[orows, :] = yrecv_buf[c].astype(jnp.float32)

        LAG = 2
        for c in range(N_CHUNKS):
            compute_send(c)
            if c >= LAG:
                reduce_forward(c - LAG)
        for c in range(N_CHUNKS - LAG, N_CHUNKS):
            reduce_forward(c)
        for c in range(N_CHUNKS):
            store_other_half(c)
        for c in range(N_CHUNKS):
            x_rdma(c).wait_send()
            y_rdma(c).wait_send()

    return pl.pallas_call(
        body,
        out_shape=jax.ShapeDtypeStruct((M, N), jnp.float32),
        in_specs=[
            pl.BlockSpec(memory_space=pltpu.VMEM),
            pl.BlockSpec(memory_space=pltpu.VMEM),
        ],
        out_specs=pl.BlockSpec(memory_space=pltpu.VMEM),
        scratch_shapes=[
            pltpu.VMEM((Ks, N), jnp.bfloat16),
            pltpu.VMEM((N_CHUNKS, CH, N), jnp.bfloat16),
            pltpu.VMEM((N_CHUNKS, CH, N), jnp.bfloat16),
            pltpu.VMEM((N_CHUNKS, CH, N), jnp.bfloat16),
            pltpu.VMEM((N_CHUNKS, CH, N), jnp.bfloat16),
            pltpu.SemaphoreType.DMA((N_CHUNKS,)),
            pltpu.SemaphoreType.DMA((N_CHUNKS,)),
            pltpu.SemaphoreType.DMA((N_CHUNKS,)),
            pltpu.SemaphoreType.DMA((N_CHUNKS,)),
        ],
        compiler_params=pltpu.CompilerParams(collective_id=0),
    )(A, B)
